# baseline (speedup 1.0000x reference)
import functools

import jax
import jax.numpy as jnp
from jax.experimental import pallas as pl
from jax.experimental.pallas import tpu as pltpu


def _loss_kernel(tgt_s_ref, x_ref, centers_ref, exc_ref, tgt_ref, out_ref,
                 xt_s, s1_acc, s2_acc, nce_s, *, n_steps, blk, k_per_group,
                 n_groups, inv_tau):
    i = pl.program_id(0)

    @pl.when(i == 0)
    def _prologue():
        s1_acc[:, :] = jnp.zeros_like(s1_acc)
        s2_acc[:, :] = jnp.zeros_like(s2_acc)
        x = x_ref[:, :]
        xt_s[:, :] = x.T
        co = jax.lax.dot_general(
            centers_ref[:, :], x,
            dimension_numbers=(((1,), (1,)), ((), ())),
            preferred_element_type=jnp.float32)
        se = jnp.sum(jnp.exp(co * inv_tau), axis=0)
        tgt = tgt_ref[0, :]
        rows = jax.lax.broadcasted_iota(jnp.int32, co.shape, 0)
        onehot = rows == tgt[None, :]
        out_t = jnp.sum(jnp.where(onehot, co, 0.0), axis=0)
        nce_s[0, 0] = -jnp.mean(out_t * inv_tau - jnp.log(se))

    eb = jax.lax.dot_general(
        exc_ref[:, :], xt_s[:, :],
        dimension_numbers=(((1,), (0,)), ((), ())),
        preferred_element_type=jnp.float32)
    ee = jnp.exp(eb * inv_tau)

    row = i * blk + jax.lax.broadcasted_iota(jnp.int32, ee.shape, 0)
    row_cluster = row // k_per_group
    member = row_cluster == tgt_s_ref[0]
    for g in range(1, n_groups):
        member = member | (row_cluster == tgt_s_ref[g * k_per_group])

    s2_acc[:, :] += jnp.sum(ee, axis=0, keepdims=True)
    s1_acc[:, :] += jnp.sum(jnp.where(member, ee, 0.0), axis=0, keepdims=True)

    @pl.when(i == n_steps - 1)
    def _finalize():
        l2 = jnp.mean(jnp.log(s2_acc[0, :]) - jnp.log(s1_acc[0, :]))
        out_ref[0, 0] = nce_s[0, 0] + l2


def kernel(inputs, idxs, targets, cams, centers, excenters):
    del idxs, cams
    b, d = inputs.shape
    c = centers.shape[0]
    _, k, _ = excenters.shape
    n_groups = b // k
    ck = excenters.shape[0] * k

    blk = 512
    n_steps = ck // blk

    exc2d = excenters.reshape(ck, d)
    tgt2d = targets.reshape(1, b)

    grid_spec = pltpu.PrefetchScalarGridSpec(
        num_scalar_prefetch=1,
        grid=(n_steps,),
        in_specs=[
            pl.BlockSpec((b, d), lambda i, g: (0, 0)),
            pl.BlockSpec((c, d), lambda i, g: (0, 0)),
            pl.BlockSpec((blk, d), lambda i, g: (i, 0)),
            pl.BlockSpec((1, b), lambda i, g: (0, 0)),
        ],
        out_specs=pl.BlockSpec(memory_space=pltpu.SMEM),
        scratch_shapes=[
            pltpu.VMEM((d, b), jnp.float32),
            pltpu.VMEM((1, b), jnp.float32),
            pltpu.VMEM((1, b), jnp.float32),
            pltpu.SMEM((1, 1), jnp.float32),
        ],
    )

    fn = functools.partial(
        _loss_kernel, n_steps=n_steps, blk=blk, k_per_group=k,
        n_groups=n_groups, inv_tau=20.0)

    out = pl.pallas_call(
        fn,
        grid_spec=grid_spec,
        out_shape=jax.ShapeDtypeStruct((1, 1), jnp.float32),
    )(targets, inputs, centers, exc2d, tgt2d)
    return out[0, 0]


# speedup vs baseline: 1.2182x; 1.2182x over previous
import functools

import jax
import jax.numpy as jnp
from jax.experimental import pallas as pl
from jax.experimental.pallas import tpu as pltpu


def _loss_kernel(tgt_s_ref, x_ref, centers_ref, exc_ref, tgt_ref, out_ref,
                 xt_s, s1_acc, s2_acc, nce_s, *, n_steps, blk, k_per_group,
                 n_groups, inv_tau):
    i = pl.program_id(0)

    @pl.when(i == 0)
    def _prologue():
        s1_acc[:, :] = jnp.zeros_like(s1_acc)
        s2_acc[:, :] = jnp.zeros_like(s2_acc)
        x = x_ref[:, :]
        xt_s[:, :] = x.T
        co = jax.lax.dot_general(
            centers_ref[:, :], x,
            dimension_numbers=(((1,), (1,)), ((), ())),
            preferred_element_type=jnp.float32)
        se = jnp.sum(jnp.exp(co * inv_tau), axis=0)
        tgt = tgt_ref[0, :]
        rows = jax.lax.broadcasted_iota(jnp.int32, co.shape, 0)
        onehot = rows == tgt[None, :]
        out_t = jnp.sum(jnp.where(onehot, co, 0.0), axis=0)
        nce_s[0, 0] = -jnp.mean(out_t * inv_tau - jnp.log(se))

    eb = jax.lax.dot_general(
        exc_ref[:, :], xt_s[:, :],
        dimension_numbers=(((1,), (0,)), ((), ())),
        preferred_element_type=jnp.float32)
    ee = jnp.exp(eb * inv_tau)

    row = i * blk + jax.lax.broadcasted_iota(jnp.int32, ee.shape, 0)
    row_cluster = row // k_per_group
    member = row_cluster == tgt_s_ref[0]
    for g in range(1, n_groups):
        member = member | (row_cluster == tgt_s_ref[g * k_per_group])

    s2_acc[:, :] += jnp.sum(ee, axis=0, keepdims=True)
    s1_acc[:, :] += jnp.sum(jnp.where(member, ee, 0.0), axis=0, keepdims=True)

    @pl.when(i == n_steps - 1)
    def _finalize():
        l2 = jnp.mean(jnp.log(s2_acc[0, :]) - jnp.log(s1_acc[0, :]))
        out_ref[0, 0] = nce_s[0, 0] + l2


def kernel(inputs, idxs, targets, cams, centers, excenters):
    del idxs, cams
    b, d = inputs.shape
    c = centers.shape[0]
    _, k, _ = excenters.shape
    n_groups = b // k
    ck = excenters.shape[0] * k

    blk = 2048
    n_steps = ck // blk

    exc2d = excenters.reshape(ck, d)
    tgt2d = targets.reshape(1, b)

    grid_spec = pltpu.PrefetchScalarGridSpec(
        num_scalar_prefetch=1,
        grid=(n_steps,),
        in_specs=[
            pl.BlockSpec((b, d), lambda i, g: (0, 0)),
            pl.BlockSpec((c, d), lambda i, g: (0, 0)),
            pl.BlockSpec((blk, d), lambda i, g: (i, 0)),
            pl.BlockSpec((1, b), lambda i, g: (0, 0)),
        ],
        out_specs=pl.BlockSpec(memory_space=pltpu.SMEM),
        scratch_shapes=[
            pltpu.VMEM((d, b), jnp.float32),
            pltpu.VMEM((1, b), jnp.float32),
            pltpu.VMEM((1, b), jnp.float32),
            pltpu.SMEM((1, 1), jnp.float32),
        ],
    )

    fn = functools.partial(
        _loss_kernel, n_steps=n_steps, blk=blk, k_per_group=k,
        n_groups=n_groups, inv_tau=20.0)

    out = pl.pallas_call(
        fn,
        grid_spec=grid_spec,
        out_shape=jax.ShapeDtypeStruct((1, 1), jnp.float32),
    )(targets, inputs, centers, exc2d, tgt2d)
    return out[0, 0]
